# baseline (speedup 1.0000x reference)
import jax
import jax.numpy as jnp
from jax.experimental import pallas as pl
from jax.experimental.pallas import tpu as pltpu

_NB = 1024
_D = 256
_EPS = 1e-07
_BN = 2


def _vq_kernel(x_ref, cb_ref, cbm2_ref, xout_ref, idx_ref, commit_ref,
               ppl_ref, cnorm_acc, iota_acc, ones_acc, ohsum_acc, commit_acc):
    i = pl.program_id(0)
    n = pl.num_programs(0)
    cb = cb_ref[...]
    cbm2 = cbm2_ref[...]

    @pl.when(i == 0)
    def _prep():
        cnorm_acc[...] = jnp.sum(cb * cb, axis=1, keepdims=True)
        iota_acc[...] = jax.lax.broadcasted_iota(
            jnp.int32, iota_acc.shape, 0).astype(jnp.float32)
        ones_acc[...] = jnp.ones_like(ones_acc)
        ohsum_acc[...] = jnp.zeros_like(ohsum_acc)
        commit_acc[0, 0] = 0.0

    cnorm = cnorm_acc[...]
    iota = iota_acc[...]

    for b in range(_BN):
        xblk = x_ref[b]
        mmn = jax.lax.dot_general(cbm2, xblk, (((1,), (0,)), ((), ())),
                                  preferred_element_type=jnp.float32)
        xnorm = jnp.sum(xblk * xblk, axis=0, keepdims=True)
        dist = (xnorm + mmn) + cnorm

        minval = jnp.min(dist, axis=0, keepdims=True)
        idx_f = jnp.min(jnp.where(dist == minval, iota, float(_NB)),
                        axis=0, keepdims=True)
        idx_ref[b] = idx_f.astype(jnp.int32)

        onehot = (iota == idx_f).astype(jnp.float32)
        xo = jax.lax.dot_general(cb, onehot, (((0,), (0,)), ((), ())),
                                 preferred_element_type=jnp.float32)
        xout_ref[b] = xo

        commit_acc[0, 0] = commit_acc[0, 0] + jnp.sum(minval)
        ohsum_acc[...] = ohsum_acc[...] + onehot

    @pl.when(i == n - 1)
    def _final():
        counts = jax.lax.dot_general(
            ohsum_acc[...], ones_acc[...], (((1,), (0,)), ((), ())),
            preferred_element_type=jnp.float32)[:, :1]
        total = jnp.sum(counts)
        prob = counts / total
        ppl = jnp.exp(-jnp.sum(prob * jnp.log(prob + _EPS)))
        ppl_ref[0, 0] = ppl
        commit_ref[0, 0] = commit_acc[0, 0] / (total * _D)


def kernel(x, codebook):
    N, D, T = x.shape
    grid = (N // _BN,)
    out_shapes = (
        jax.ShapeDtypeStruct((N, D, T), jnp.float32),
        jax.ShapeDtypeStruct((N, 1, T), jnp.int32),
        jax.ShapeDtypeStruct((1, 1), jnp.float32),
        jax.ShapeDtypeStruct((1, 1), jnp.float32),
    )
    cbm2 = -2.0 * codebook
    x_out, idx, commit, ppl = pl.pallas_call(
        _vq_kernel,
        grid=grid,
        in_specs=[
            pl.BlockSpec((_BN, D, T), lambda i: (i, 0, 0)),
            pl.BlockSpec((_NB, _D), lambda i: (0, 0)),
            pl.BlockSpec((_NB, _D), lambda i: (0, 0)),
        ],
        out_specs=(
            pl.BlockSpec((_BN, D, T), lambda i: (i, 0, 0)),
            pl.BlockSpec((_BN, 1, T), lambda i: (i, 0, 0)),
            pl.BlockSpec(memory_space=pltpu.SMEM),
            pl.BlockSpec(memory_space=pltpu.SMEM),
        ),
        out_shape=out_shapes,
        scratch_shapes=[
            pltpu.VMEM((_NB, 1), jnp.float32),
            pltpu.VMEM((_NB, T), jnp.float32),
            pltpu.VMEM((T, 128), jnp.float32),
            pltpu.VMEM((_NB, T), jnp.float32),
            pltpu.SMEM((1, 1), jnp.float32),
        ],
    )(x, codebook, cbm2)
    return (x_out,
            idx.reshape(N, T),
            commit.reshape(()),
            ppl.reshape(()))


# speedup vs baseline: 1.0251x; 1.0251x over previous
import jax
import jax.numpy as jnp
from jax.experimental import pallas as pl
from jax.experimental.pallas import tpu as pltpu

_NB = 1024
_D = 256
_EPS = 1e-07
_BN = 8


def _vq_kernel(x_ref, cb_ref, cbm2_ref, xout_ref, idx_ref, commit_ref,
               ppl_ref, cnorm_acc, iota_acc, ones_acc, ohsum_acc, commit_acc):
    i = pl.program_id(0)
    n = pl.num_programs(0)
    cb = cb_ref[...]
    cbm2 = cbm2_ref[...]

    @pl.when(i == 0)
    def _prep():
        cnorm_acc[...] = jnp.sum(cb * cb, axis=1, keepdims=True)
        iota_acc[...] = jax.lax.broadcasted_iota(
            jnp.int32, iota_acc.shape, 0).astype(jnp.float32)
        ones_acc[...] = jnp.ones_like(ones_acc)
        ohsum_acc[...] = jnp.zeros_like(ohsum_acc)
        commit_acc[0, 0] = 0.0

    cnorm = cnorm_acc[...]
    iota = iota_acc[...]

    for b in range(_BN):
        xblk = x_ref[b]
        mmn = jax.lax.dot_general(cbm2, xblk, (((1,), (0,)), ((), ())),
                                  preferred_element_type=jnp.float32)
        xnorm = jnp.sum(xblk * xblk, axis=0, keepdims=True)
        dist = (xnorm + mmn) + cnorm

        minval = jnp.min(dist, axis=0, keepdims=True)
        idx_f = jnp.min(jnp.where(dist == minval, iota, float(_NB)),
                        axis=0, keepdims=True)
        idx_ref[b] = idx_f.astype(jnp.int32)

        onehot = (iota == idx_f).astype(jnp.float32)
        xo = jax.lax.dot_general(cb, onehot, (((0,), (0,)), ((), ())),
                                 preferred_element_type=jnp.float32)
        xout_ref[b] = xo

        commit_acc[0, 0] = commit_acc[0, 0] + jnp.sum(minval)
        ohsum_acc[...] = ohsum_acc[...] + onehot

    @pl.when(i == n - 1)
    def _final():
        counts = jax.lax.dot_general(
            ohsum_acc[...], ones_acc[...], (((1,), (0,)), ((), ())),
            preferred_element_type=jnp.float32)[:, :1]
        total = jnp.sum(counts)
        prob = counts / total
        ppl = jnp.exp(-jnp.sum(prob * jnp.log(prob + _EPS)))
        ppl_ref[0, 0] = ppl
        commit_ref[0, 0] = commit_acc[0, 0] / (total * _D)


def kernel(x, codebook):
    N, D, T = x.shape
    grid = (N // _BN,)
    out_shapes = (
        jax.ShapeDtypeStruct((N, D, T), jnp.float32),
        jax.ShapeDtypeStruct((N, 1, T), jnp.int32),
        jax.ShapeDtypeStruct((1, 1), jnp.float32),
        jax.ShapeDtypeStruct((1, 1), jnp.float32),
    )
    cbm2 = -2.0 * codebook
    x_out, idx, commit, ppl = pl.pallas_call(
        _vq_kernel,
        grid=grid,
        in_specs=[
            pl.BlockSpec((_BN, D, T), lambda i: (i, 0, 0)),
            pl.BlockSpec((_NB, _D), lambda i: (0, 0)),
            pl.BlockSpec((_NB, _D), lambda i: (0, 0)),
        ],
        out_specs=(
            pl.BlockSpec((_BN, D, T), lambda i: (i, 0, 0)),
            pl.BlockSpec((_BN, 1, T), lambda i: (i, 0, 0)),
            pl.BlockSpec(memory_space=pltpu.SMEM),
            pl.BlockSpec(memory_space=pltpu.SMEM),
        ),
        out_shape=out_shapes,
        scratch_shapes=[
            pltpu.VMEM((_NB, 1), jnp.float32),
            pltpu.VMEM((_NB, T), jnp.float32),
            pltpu.VMEM((T, 128), jnp.float32),
            pltpu.VMEM((_NB, T), jnp.float32),
            pltpu.SMEM((1, 1), jnp.float32),
        ],
    )(x, codebook, cbm2)
    return (x_out,
            idx.reshape(N, T),
            commit.reshape(()),
            ppl.reshape(()))


# speedup vs baseline: 1.0340x; 1.0086x over previous
import jax
import jax.numpy as jnp
from jax.experimental import pallas as pl
from jax.experimental.pallas import tpu as pltpu

_NB = 1024
_D = 256
_EPS = 1e-07
_BN = 4


def _vq_kernel(x_ref, cb_ref, cbm2_ref, xout_ref, idx_ref, commit_ref,
               ppl_ref, cnorm_acc, iota_acc, ones_acc, ohsum_acc, commit_acc):
    i = pl.program_id(0)
    n = pl.num_programs(0)
    cb = cb_ref[...]
    cbm2 = cbm2_ref[...]

    @pl.when(i == 0)
    def _prep():
        cnorm_acc[...] = jnp.sum(cb * cb, axis=1, keepdims=True)
        iota_acc[...] = jax.lax.broadcasted_iota(
            jnp.int32, iota_acc.shape, 0).astype(jnp.float32)
        ones_acc[...] = jnp.ones_like(ones_acc)
        ohsum_acc[...] = jnp.zeros_like(ohsum_acc)
        commit_acc[0, 0] = 0.0

    cnorm = cnorm_acc[...]
    iota = iota_acc[...]

    for b in range(_BN):
        xblk = x_ref[b]
        mmn = jax.lax.dot_general(cbm2, xblk, (((1,), (0,)), ((), ())),
                                  preferred_element_type=jnp.float32)
        xnorm = jnp.sum(xblk * xblk, axis=0, keepdims=True)
        dist = (xnorm + mmn) + cnorm

        minval = jnp.min(dist, axis=0, keepdims=True)
        idx_f = jnp.min(jnp.where(dist == minval, iota, float(_NB)),
                        axis=0, keepdims=True)
        idx_ref[b] = idx_f.astype(jnp.int32)

        onehot = (iota == idx_f).astype(jnp.float32)
        xo = jax.lax.dot_general(cb, onehot, (((0,), (0,)), ((), ())),
                                 preferred_element_type=jnp.float32)
        xout_ref[b] = xo

        commit_acc[0, 0] = commit_acc[0, 0] + jnp.sum(minval)
        ohsum_acc[...] = ohsum_acc[...] + onehot

    @pl.when(i == n - 1)
    def _final():
        counts = jax.lax.dot_general(
            ohsum_acc[...], ones_acc[...], (((1,), (0,)), ((), ())),
            preferred_element_type=jnp.float32)[:, :1]
        total = jnp.sum(counts)
        prob = counts / total
        ppl = jnp.exp(-jnp.sum(prob * jnp.log(prob + _EPS)))
        ppl_ref[0, 0] = ppl
        commit_ref[0, 0] = commit_acc[0, 0] / (total * _D)


def kernel(x, codebook):
    N, D, T = x.shape
    grid = (N // _BN,)
    out_shapes = (
        jax.ShapeDtypeStruct((N, D, T), jnp.float32),
        jax.ShapeDtypeStruct((N, 1, T), jnp.int32),
        jax.ShapeDtypeStruct((1, 1), jnp.float32),
        jax.ShapeDtypeStruct((1, 1), jnp.float32),
    )
    cbm2 = -2.0 * codebook
    x_out, idx, commit, ppl = pl.pallas_call(
        _vq_kernel,
        grid=grid,
        in_specs=[
            pl.BlockSpec((_BN, D, T), lambda i: (i, 0, 0)),
            pl.BlockSpec((_NB, _D), lambda i: (0, 0)),
            pl.BlockSpec((_NB, _D), lambda i: (0, 0)),
        ],
        out_specs=(
            pl.BlockSpec((_BN, D, T), lambda i: (i, 0, 0)),
            pl.BlockSpec((_BN, 1, T), lambda i: (i, 0, 0)),
            pl.BlockSpec(memory_space=pltpu.SMEM),
            pl.BlockSpec(memory_space=pltpu.SMEM),
        ),
        out_shape=out_shapes,
        scratch_shapes=[
            pltpu.VMEM((_NB, 1), jnp.float32),
            pltpu.VMEM((_NB, T), jnp.float32),
            pltpu.VMEM((T, 128), jnp.float32),
            pltpu.VMEM((_NB, T), jnp.float32),
            pltpu.SMEM((1, 1), jnp.float32),
        ],
    )(x, codebook, cbm2)
    return (x_out,
            idx.reshape(N, T),
            commit.reshape(()),
            ppl.reshape(()))


# speedup vs baseline: 1.1342x; 1.0969x over previous
import jax
import jax.numpy as jnp
from jax.experimental import pallas as pl
from jax.experimental.pallas import tpu as pltpu

_NB = 1024
_D = 256
_EPS = 1e-07
_BN = 4


def _vq_kernel(x_ref, cb_ref, xout_ref, idx_ref, commit_ref,
               ppl_ref, cnorm_acc, iota_acc, ones_acc, ohsum_acc, cbm2_acc,
               commit_acc):
    i = pl.program_id(0)
    n = pl.num_programs(0)
    cb = cb_ref[...]

    @pl.when(i == 0)
    def _prep():
        cbm2_acc[...] = -2.0 * cb
        cnorm_acc[...] = jnp.sum(cb * cb, axis=1, keepdims=True)
        iota_acc[...] = jax.lax.broadcasted_iota(
            jnp.int32, iota_acc.shape, 0).astype(jnp.float32)
        ones_acc[...] = jnp.ones_like(ones_acc)
        ohsum_acc[...] = jnp.zeros_like(ohsum_acc)
        commit_acc[0, 0] = 0.0

    cnorm = cnorm_acc[...]
    iota = iota_acc[...]
    cbm2 = cbm2_acc[...]

    for b in range(_BN):
        xblk = x_ref[b]
        mmn = jax.lax.dot_general(cbm2, xblk, (((1,), (0,)), ((), ())),
                                  preferred_element_type=jnp.float32)
        xnorm = jnp.sum(xblk * xblk, axis=0, keepdims=True)
        dist = (xnorm + mmn) + cnorm

        minval = jnp.min(dist, axis=0, keepdims=True)
        idx_f = jnp.min(jnp.where(dist == minval, iota, float(_NB)),
                        axis=0, keepdims=True)
        idx_ref[b] = idx_f.astype(jnp.int32)

        onehot = (iota == idx_f).astype(jnp.float32)
        xo = jax.lax.dot_general(cb, onehot, (((0,), (0,)), ((), ())),
                                 preferred_element_type=jnp.float32)
        xout_ref[b] = xo

        commit_acc[0, 0] = commit_acc[0, 0] + jnp.sum(minval)
        ohsum_acc[...] = ohsum_acc[...] + onehot

    @pl.when(i == n - 1)
    def _final():
        counts = jax.lax.dot_general(
            ohsum_acc[...], ones_acc[...], (((1,), (0,)), ((), ())),
            preferred_element_type=jnp.float32)[:, :1]
        total = jnp.sum(counts)
        prob = counts / total
        ppl = jnp.exp(-jnp.sum(prob * jnp.log(prob + _EPS)))
        ppl_ref[0, 0] = ppl
        commit_ref[0, 0] = commit_acc[0, 0] / (total * _D)


def kernel(x, codebook):
    N, D, T = x.shape
    grid = (N // _BN,)
    out_shapes = (
        jax.ShapeDtypeStruct((N, D, T), jnp.float32),
        jax.ShapeDtypeStruct((N, 1, T), jnp.int32),
        jax.ShapeDtypeStruct((1, 1), jnp.float32),
        jax.ShapeDtypeStruct((1, 1), jnp.float32),
    )
    x_out, idx, commit, ppl = pl.pallas_call(
        _vq_kernel,
        grid=grid,
        in_specs=[
            pl.BlockSpec((_BN, D, T), lambda i: (i, 0, 0)),
            pl.BlockSpec((_NB, _D), lambda i: (0, 0)),
        ],
        out_specs=(
            pl.BlockSpec((_BN, D, T), lambda i: (i, 0, 0)),
            pl.BlockSpec((_BN, 1, T), lambda i: (i, 0, 0)),
            pl.BlockSpec(memory_space=pltpu.SMEM),
            pl.BlockSpec(memory_space=pltpu.SMEM),
        ),
        out_shape=out_shapes,
        scratch_shapes=[
            pltpu.VMEM((_NB, 1), jnp.float32),
            pltpu.VMEM((_NB, T), jnp.float32),
            pltpu.VMEM((T, 128), jnp.float32),
            pltpu.VMEM((_NB, T), jnp.float32),
            pltpu.VMEM((_NB, _D), jnp.float32),
            pltpu.SMEM((1, 1), jnp.float32),
        ],
    )(x, codebook)
    return (x_out,
            idx.reshape(N, T),
            commit.reshape(()),
            ppl.reshape(()))
